# baseline (speedup 1.0000x reference)
import functools

import jax
import jax.numpy as jnp
from jax import lax
from jax.experimental import pallas as pl
from jax.experimental.pallas import tpu as pltpu
from jax.experimental.pallas import tpu_sc as plsc

N_NODES = 10000
N_EDGES = 320000
D = 128
NC = 2
NS = 16
NW = NC * NS
NPAD = 10240
RPT = NPAD // NS
CHUNK = 80
CPT = N_EDGES // CHUNK // NW
NB = 5
BCH = CPT // NB
LANES = 16


def _make_sc_agg(with_cnt: bool):
    out_type = [jax.ShapeDtypeStruct((NC, NPAD, D), jnp.float32)]
    if with_cnt:
        out_type.append(jax.ShapeDtypeStruct((NPAD,), jnp.float32))
        out_type.append(jax.ShapeDtypeStruct((NPAD,), jnp.float32))

    scratch = [
        pltpu.VMEM((BCH, CHUNK), jnp.int32),
        pltpu.VMEM((BCH, CHUNK), jnp.int32),
        pltpu.VMEM((BCH, CHUNK), jnp.int32),
        pltpu.VMEM((BCH, CHUNK), jnp.int32),
        pltpu.VMEM((CHUNK, D), jnp.float32),
        pltpu.VMEM((CHUNK, D), jnp.float32),
        pltpu.VMEM_SHARED((NPAD, D), jnp.float32),
        pltpu.SemaphoreType.DMA,
        pltpu.SemaphoreType.DMA,
        pltpu.SemaphoreType.DMA,
        pltpu.SemaphoreType.DMA,
    ]
    if with_cnt:
        scratch += [
            pltpu.VMEM((RPT,), jnp.float32),
            pltpu.VMEM((CHUNK,), jnp.float32),
            pltpu.VMEM_SHARED((NPAD,), jnp.float32),
        ]

    def body(x_hbm, s4_hbm, d4_hbm, part_out, *rest):
        if with_cnt:
            (cnt_out0, cnt_out1, sidxA, didxA, sidxB, didxB, rows0, rows1,
             acc, sem0, sem1, semiA, semiB, zc, ones, cnt_sh) = rest
        else:
            (sidxA, didxA, sidxB, didxB, rows0, rows1,
             acc, sem0, sem1, semiA, semiB) = rest
        cid = lax.axis_index("c")
        sid = lax.axis_index("s")
        w = sid * NC + cid
        zero16 = jnp.zeros((LANES,), jnp.float32)

        pltpu.async_copy(s4_hbm.at[w, 0], sidxA, semiA)
        pltpu.async_copy(d4_hbm.at[w, 0], didxA, semiA)

        def zrow(i, carry):
            for j in range(D // LANES):
                rows0[i, pl.ds(j * LANES, LANES)] = zero16
            return carry
        lax.fori_loop(0, CHUNK, zrow, 0)
        rb = sid * RPT
        for k in range(RPT // CHUNK):
            pltpu.sync_copy(rows0, acc.at[pl.ds(rb + k * CHUNK, CHUNK)])
        if with_cnt:
            def zrow2(i, carry):
                zc[pl.ds(i * LANES, LANES)] = zero16
                return carry
            lax.fori_loop(0, RPT // LANES, zrow2, 0)
            for j in range(CHUNK // LANES):
                ones[pl.ds(j * LANES, LANES)] = jnp.ones((LANES,), jnp.float32)
            pltpu.sync_copy(zc, cnt_sh.at[pl.ds(rb, RPT)])
        plsc.subcore_barrier()

        for b in range(NB):
            si, di, isem = ((sidxA, didxA, semiA) if b % 2 == 0
                            else (sidxB, didxB, semiB))
            pltpu.make_async_copy(s4_hbm.at[w, b], si, isem).wait()
            pltpu.make_async_copy(d4_hbm.at[w, b], di, isem).wait()
            if b + 1 < NB:
                nsi, ndi, nisem = ((sidxB, didxB, semiB) if b % 2 == 0
                                   else (sidxA, didxA, semiA))
                pltpu.async_copy(s4_hbm.at[w, b + 1], nsi, nisem)
                pltpu.async_copy(d4_hbm.at[w, b + 1], ndi, nisem)

            def scat(c, buf, di=di):
                pltpu.sync_copy(buf, acc.at[di.at[c]], add=True)
                if with_cnt:
                    pltpu.sync_copy(ones, cnt_sh.at[di.at[c]], add=True)

            pltpu.async_copy(x_hbm.at[si.at[0]], rows0, sem0)

            def pair_body(t, carry, si=si, scat=scat):
                c0 = 2 * t
                d1 = pltpu.async_copy(x_hbm.at[si.at[c0 + 1]], rows1, sem1)
                pltpu.make_async_copy(x_hbm.at[si.at[c0]], rows0, sem0).wait()
                scat(c0, rows0)
                pltpu.async_copy(x_hbm.at[si.at[c0 + 2]], rows0, sem0)
                d1.wait()
                scat(c0 + 1, rows1)
                return carry
            lax.fori_loop(0, (BCH - 1) // 2, pair_body, 0)
            pltpu.make_async_copy(x_hbm.at[si.at[BCH - 1]], rows0, sem0).wait()
            scat(BCH - 1, rows0)

        plsc.subcore_barrier()

        pltpu.sync_copy(acc.at[pl.ds(rb, RPT)], part_out.at[cid, pl.ds(rb, RPT)])
        if with_cnt:
            @pl.when(cid == 0)
            def _():
                pltpu.sync_copy(cnt_sh.at[pl.ds(rb, RPT)], cnt_out0.at[pl.ds(rb, RPT)])

            @pl.when(cid == 1)
            def _():
                pltpu.sync_copy(cnt_sh.at[pl.ds(rb, RPT)], cnt_out1.at[pl.ds(rb, RPT)])

    mesh = plsc.VectorSubcoreMesh(core_axis_name="c", subcore_axis_name="s")
    return pl.kernel(body, out_type=tuple(out_type), mesh=mesh,
                     scratch_types=scratch)


_sc_agg_cnt = _make_sc_agg(with_cnt=True)
_sc_agg = _make_sc_agg(with_cnt=False)


def _make_dense(with_relu: bool):
    R = 1000

    def matt(a, w):
        return lax.dot_general(a, w[...], (((1,), (1,)), ((), ())),
                               preferred_element_type=jnp.float32)

    def body(p0, p1, c0, c1, xr, wl, wr, br, o):
        cnt = jnp.maximum(c0[...] + c1[...], 1.0)
        agg = (p0[0] + p1[0]) / cnt
        r = matt(agg, wl) + br[...] + matt(xr[...], wr)
        if with_relu:
            r = jnp.maximum(r, 0.0)
        o[...] = r

    row_spec = pl.BlockSpec((R, D), lambda i: (i, 0))
    p0_spec = pl.BlockSpec((1, R, D), lambda i: (0, i, 0))
    p1_spec = pl.BlockSpec((1, R, D), lambda i: (1, i, 0))
    col_spec = pl.BlockSpec((R, 1), lambda i: (i, 0))
    w_spec = pl.BlockSpec((D, D), lambda i: (0, 0))
    b_spec = pl.BlockSpec((1, D), lambda i: (0, 0))
    return pl.pallas_call(
        body,
        grid=(N_NODES // R,),
        in_specs=[p0_spec, p1_spec, col_spec, col_spec, row_spec,
                  w_spec, w_spec, b_spec],
        out_specs=row_spec,
        out_shape=jax.ShapeDtypeStruct((N_NODES, D), jnp.float32),
    )


_dense_relu = _make_dense(with_relu=True)
_dense = _make_dense(with_relu=False)


def kernel(x, edge_index, W1l, b1, W1r, W2l, b2, W2r):
    ei = edge_index.astype(jnp.int32)
    src4d = ei[0].reshape(NW, NB, BCH, CHUNK)
    dst4d = ei[1].reshape(NW, NB, BCH, CHUNK)

    part1, cnt0_f, cnt1_f = _sc_agg_cnt(x, src4d, dst4d)
    c0 = cnt0_f.reshape(NPAD, 1)
    c1 = cnt1_f.reshape(NPAD, 1)
    h = _dense_relu(part1, part1, c0, c1, x, W1l, W1r, b1.reshape(1, D))

    (part2,) = _sc_agg(h, src4d, dst4d)
    out = _dense(part2, part2, c0, c1, h, W2l, W2r, b2.reshape(1, D))
    return out


# speedup vs baseline: 1.0035x; 1.0035x over previous
import functools

import jax
import jax.numpy as jnp
from jax import lax
from jax.experimental import pallas as pl
from jax.experimental.pallas import tpu as pltpu
from jax.experimental.pallas import tpu_sc as plsc

N_NODES = 10000
N_EDGES = 320000
D = 128
NC = 2
NS = 16
NW = NC * NS
NPAD = 10240
RPT = NPAD // NS
CHUNK = 80
CPT = N_EDGES // CHUNK // NW
NB = 5
BCH = CPT // NB
LANES = 16


def _make_sc_agg(with_cnt: bool):
    out_type = [jax.ShapeDtypeStruct((NC, NPAD, D), jnp.float32)]
    if with_cnt:
        out_type.append(jax.ShapeDtypeStruct((NPAD,), jnp.float32))
        out_type.append(jax.ShapeDtypeStruct((NPAD,), jnp.float32))

    scratch = [
        pltpu.VMEM((BCH, CHUNK), jnp.int32),
        pltpu.VMEM((BCH, CHUNK), jnp.int32),
        pltpu.VMEM((BCH, CHUNK), jnp.int32),
        pltpu.VMEM((BCH, CHUNK), jnp.int32),
        pltpu.VMEM((CHUNK, D), jnp.float32),
        pltpu.VMEM((CHUNK, D), jnp.float32),
        pltpu.VMEM_SHARED((NPAD, D), jnp.float32),
        pltpu.SemaphoreType.DMA,
        pltpu.SemaphoreType.DMA,
        pltpu.SemaphoreType.DMA,
        pltpu.SemaphoreType.DMA,
    ]
    if with_cnt:
        scratch += [
            pltpu.VMEM((RPT,), jnp.float32),
            pltpu.VMEM((CHUNK,), jnp.float32),
            pltpu.VMEM_SHARED((NPAD,), jnp.float32),
        ]

    def body(x_hbm, s4_hbm, d4_hbm, part_out, *rest):
        if with_cnt:
            (cnt_out0, cnt_out1, sidxA, didxA, sidxB, didxB, rows0, rows1,
             acc, sem0, sem1, semiA, semiB, zc, ones, cnt_sh) = rest
        else:
            (sidxA, didxA, sidxB, didxB, rows0, rows1,
             acc, sem0, sem1, semiA, semiB) = rest
        cid = lax.axis_index("c")
        sid = lax.axis_index("s")
        w = sid * NC + cid
        zero16 = jnp.zeros((LANES,), jnp.float32)

        pltpu.async_copy(s4_hbm.at[w, 0], sidxA, semiA)
        pltpu.async_copy(d4_hbm.at[w, 0], didxA, semiA)

        def zrow(i, carry):
            for j in range(D // LANES):
                rows0[i, pl.ds(j * LANES, LANES)] = zero16
            return carry
        lax.fori_loop(0, CHUNK, zrow, 0)
        rb = sid * RPT
        for k in range(RPT // CHUNK):
            pltpu.sync_copy(rows0, acc.at[pl.ds(rb + k * CHUNK, CHUNK)])
        if with_cnt:
            def zrow2(i, carry):
                zc[pl.ds(i * LANES, LANES)] = zero16
                return carry
            lax.fori_loop(0, RPT // LANES, zrow2, 0)
            for j in range(CHUNK // LANES):
                ones[pl.ds(j * LANES, LANES)] = jnp.ones((LANES,), jnp.float32)
            pltpu.sync_copy(zc, cnt_sh.at[pl.ds(rb, RPT)])
        plsc.subcore_barrier()

        for b in range(NB):
            si, di, isem = ((sidxA, didxA, semiA) if b % 2 == 0
                            else (sidxB, didxB, semiB))
            pltpu.make_async_copy(s4_hbm.at[w, b], si, isem).wait()
            pltpu.make_async_copy(d4_hbm.at[w, b], di, isem).wait()
            if b + 1 < NB:
                nsi, ndi, nisem = ((sidxB, didxB, semiB) if b % 2 == 0
                                   else (sidxA, didxA, semiA))
                pltpu.async_copy(s4_hbm.at[w, b + 1], nsi, nisem)
                pltpu.async_copy(d4_hbm.at[w, b + 1], ndi, nisem)

            def scat(c, buf, di=di):
                pltpu.sync_copy(buf, acc.at[di.at[c]], add=True)
                if with_cnt:
                    pltpu.sync_copy(ones, cnt_sh.at[di.at[c]], add=True)

            pltpu.async_copy(x_hbm.at[si.at[0]], rows0, sem0)

            def pair_body(t, carry, si=si, scat=scat):
                c0 = 2 * t
                d1 = pltpu.async_copy(x_hbm.at[si.at[c0 + 1]], rows1, sem1)
                pltpu.make_async_copy(x_hbm.at[si.at[c0]], rows0, sem0).wait()
                scat(c0, rows0)
                pltpu.async_copy(x_hbm.at[si.at[c0 + 2]], rows0, sem0)
                d1.wait()
                scat(c0 + 1, rows1)
                return carry
            lax.fori_loop(0, (BCH - 1) // 2, pair_body, 0)
            pltpu.make_async_copy(x_hbm.at[si.at[BCH - 1]], rows0, sem0).wait()
            scat(BCH - 1, rows0)

        plsc.subcore_barrier()

        pltpu.sync_copy(acc.at[pl.ds(rb, RPT)], part_out.at[cid, pl.ds(rb, RPT)])
        if with_cnt:
            @pl.when(cid == 0)
            def _():
                pltpu.sync_copy(cnt_sh.at[pl.ds(rb, RPT)], cnt_out0.at[pl.ds(rb, RPT)])

            @pl.when(cid == 1)
            def _():
                pltpu.sync_copy(cnt_sh.at[pl.ds(rb, RPT)], cnt_out1.at[pl.ds(rb, RPT)])

    mesh = plsc.VectorSubcoreMesh(core_axis_name="c", subcore_axis_name="s")
    return pl.kernel(body, out_type=tuple(out_type), mesh=mesh,
                     scratch_types=scratch)


_sc_agg_cnt = _make_sc_agg(with_cnt=True)
_sc_agg = _make_sc_agg(with_cnt=False)


def _make_dense(with_relu: bool):
    R = 1000

    def matt(a, w):
        return lax.dot_general(a, w[...], (((1,), (1,)), ((), ())),
                               preferred_element_type=jnp.float32)

    def body(p0, p1, c0, c1, sf, wl, o):
        cnt = jnp.maximum(c0[...] + c1[...], 1.0)
        agg = (p0[0] + p1[0]) / cnt
        r = matt(agg, wl) + sf[...]
        if with_relu:
            r = jnp.maximum(r, 0.0)
        o[...] = r

    row_spec = pl.BlockSpec((R, D), lambda i: (i, 0))
    p0_spec = pl.BlockSpec((1, R, D), lambda i: (0, i, 0))
    p1_spec = pl.BlockSpec((1, R, D), lambda i: (1, i, 0))
    col_spec = pl.BlockSpec((R, 1), lambda i: (i, 0))
    w_spec = pl.BlockSpec((D, D), lambda i: (0, 0))
    return pl.pallas_call(
        body,
        grid=(N_NODES // R,),
        in_specs=[p0_spec, p1_spec, col_spec, col_spec, row_spec, w_spec],
        out_specs=row_spec,
        out_shape=jax.ShapeDtypeStruct((N_NODES, D), jnp.float32),
    )


def _self_mm(xin, wr, br):
    R = 2000

    def body(xr, w, b, o):
        o[...] = lax.dot_general(xr[...], w[...], (((1,), (1,)), ((), ())),
                                 preferred_element_type=jnp.float32) + b[...]

    return pl.pallas_call(
        body,
        grid=(N_NODES // R,),
        in_specs=[pl.BlockSpec((R, D), lambda i: (i, 0)),
                  pl.BlockSpec((D, D), lambda i: (0, 0)),
                  pl.BlockSpec((1, D), lambda i: (0, 0))],
        out_specs=pl.BlockSpec((R, D), lambda i: (i, 0)),
        out_shape=jax.ShapeDtypeStruct((N_NODES, D), jnp.float32),
    )(xin, wr, br)


_dense_relu = _make_dense(with_relu=True)
_dense = _make_dense(with_relu=False)


def kernel(x, edge_index, W1l, b1, W1r, W2l, b2, W2r):
    ei = edge_index.astype(jnp.int32)
    src4d = ei[0].reshape(NW, NB, BCH, CHUNK)
    dst4d = ei[1].reshape(NW, NB, BCH, CHUNK)

    part1, cnt0_f, cnt1_f = _sc_agg_cnt(x, src4d, dst4d)
    self1 = _self_mm(x, W1r, b1.reshape(1, D))
    c0 = cnt0_f.reshape(NPAD, 1)
    c1 = cnt1_f.reshape(NPAD, 1)
    h = _dense_relu(part1, part1, c0, c1, self1, W1l)

    (part2,) = _sc_agg(h, src4d, dst4d)
    self2 = _self_mm(h, W2r, b2.reshape(1, D))
    out = _dense(part2, part2, c0, c1, self2, W2l)
    return out


# speedup vs baseline: 1.1207x; 1.1168x over previous
import functools

import jax
import jax.numpy as jnp
from jax import lax
from jax.experimental import pallas as pl
from jax.experimental.pallas import tpu as pltpu
from jax.experimental.pallas import tpu_sc as plsc

N_NODES = 10000
N_EDGES = 320000
D = 128
NC = 2
NS = 16
NW = NC * NS
NPAD = 10240
RPT = NPAD // NS
CHUNK = 80
CPT = N_EDGES // CHUNK // NW
NB = 5
BCH = CPT // NB
LANES = 16


def _make_sc_agg(with_cnt: bool, nring: int = 2):
    out_type = [jax.ShapeDtypeStruct((NC, NPAD, D), jnp.float32)]
    if with_cnt:
        out_type.append(jax.ShapeDtypeStruct((NPAD,), jnp.float32))
        out_type.append(jax.ShapeDtypeStruct((NPAD,), jnp.float32))

    scratch = (
        [pltpu.VMEM((BCH, CHUNK), jnp.int32)] * 4 +
        [pltpu.VMEM((CHUNK, D), jnp.float32)] * nring +
        [pltpu.VMEM_SHARED((NPAD, D), jnp.float32)] +
        [pltpu.SemaphoreType.DMA] * (nring + 2)
    )
    if with_cnt:
        scratch += [
            pltpu.VMEM((RPT,), jnp.float32),
            pltpu.VMEM((CHUNK,), jnp.float32),
            pltpu.VMEM_SHARED((NPAD,), jnp.float32),
        ]

    def body(x_hbm, s4_hbm, d4_hbm, part_out, *rest):
        if with_cnt:
            cnt_out0, cnt_out1 = rest[:2]
            rest = rest[2:]
            zc, ones, cnt_sh = rest[-3:]
        sidxA, didxA, sidxB, didxB = rest[:4]
        bufs = rest[4:4 + nring]
        acc = rest[4 + nring]
        sems = rest[5 + nring:5 + 2 * nring]
        semiA, semiB = rest[5 + 2 * nring:7 + 2 * nring]
        rows0 = bufs[0]
        cid = lax.axis_index("c")
        sid = lax.axis_index("s")
        w = sid * NC + cid
        zero16 = jnp.zeros((LANES,), jnp.float32)

        pltpu.async_copy(s4_hbm.at[w, 0], sidxA, semiA)
        pltpu.async_copy(d4_hbm.at[w, 0], didxA, semiA)

        def zrow(i, carry):
            for j in range(D // LANES):
                rows0[i, pl.ds(j * LANES, LANES)] = zero16
            return carry
        lax.fori_loop(0, CHUNK, zrow, 0)
        rb = sid * RPT
        for k in range(RPT // CHUNK):
            pltpu.sync_copy(rows0, acc.at[pl.ds(rb + k * CHUNK, CHUNK)])
        if with_cnt:
            def zrow2(i, carry):
                zc[pl.ds(i * LANES, LANES)] = zero16
                return carry
            lax.fori_loop(0, RPT // LANES, zrow2, 0)
            for j in range(CHUNK // LANES):
                ones[pl.ds(j * LANES, LANES)] = jnp.ones((LANES,), jnp.float32)
            pltpu.sync_copy(zc, cnt_sh.at[pl.ds(rb, RPT)])
        plsc.subcore_barrier()

        for b in range(NB):
            si, di, isem = ((sidxA, didxA, semiA) if b % 2 == 0
                            else (sidxB, didxB, semiB))
            pltpu.make_async_copy(s4_hbm.at[w, b], si, isem).wait()
            pltpu.make_async_copy(d4_hbm.at[w, b], di, isem).wait()
            if b + 1 < NB:
                nsi, ndi, nisem = ((sidxB, didxB, semiB) if b % 2 == 0
                                   else (sidxA, didxA, semiA))
                pltpu.async_copy(s4_hbm.at[w, b + 1], nsi, nisem)
                pltpu.async_copy(d4_hbm.at[w, b + 1], ndi, nisem)

            def scat(c, buf, di=di):
                pltpu.sync_copy(buf, acc.at[di.at[c]], add=True)
                if with_cnt:
                    pltpu.sync_copy(ones, cnt_sh.at[di.at[c]], add=True)

            for k in range(nring):
                pltpu.async_copy(x_hbm.at[si.at[k]], bufs[k], sems[k])

            def grp_body(t, carry, si=si, scat=scat):
                base = t * nring
                for k in range(nring):
                    c = base + k
                    pltpu.make_async_copy(x_hbm.at[si.at[c]], bufs[k],
                                          sems[k]).wait()
                    scat(c, bufs[k])
                    cn = jnp.minimum(c + nring, BCH - 1)
                    pltpu.async_copy(x_hbm.at[si.at[cn]], bufs[k], sems[k])
                return carry
            lax.fori_loop(0, (BCH - 1) // nring, grp_body, 0)
            pltpu.make_async_copy(x_hbm.at[si.at[BCH - 1]], bufs[0],
                                  sems[0]).wait()
            scat(BCH - 1, bufs[0])
            for k in range(1, nring):
                pltpu.make_async_copy(x_hbm.at[si.at[BCH - 1]], bufs[k],
                                      sems[k]).wait()

        plsc.subcore_barrier()

        pltpu.sync_copy(acc.at[pl.ds(rb, RPT)], part_out.at[cid, pl.ds(rb, RPT)])
        if with_cnt:
            @pl.when(cid == 0)
            def _():
                pltpu.sync_copy(cnt_sh.at[pl.ds(rb, RPT)], cnt_out0.at[pl.ds(rb, RPT)])

            @pl.when(cid == 1)
            def _():
                pltpu.sync_copy(cnt_sh.at[pl.ds(rb, RPT)], cnt_out1.at[pl.ds(rb, RPT)])

    mesh = plsc.VectorSubcoreMesh(core_axis_name="c", subcore_axis_name="s")
    return pl.kernel(body, out_type=tuple(out_type), mesh=mesh,
                     scratch_types=scratch)


_sc_agg_cnt = _make_sc_agg(with_cnt=True, nring=3)
_sc_agg = _make_sc_agg(with_cnt=False, nring=3)


def _make_dense(with_relu: bool):
    R = 1000

    def matt(a, w):
        return lax.dot_general(a, w[...], (((1,), (1,)), ((), ())),
                               preferred_element_type=jnp.float32)

    def body(p0, p1, c0, c1, xr, wl, wr, br, o):
        cnt = jnp.maximum(c0[...] + c1[...], 1.0)
        agg = (p0[0] + p1[0]) / cnt
        r = matt(agg, wl) + br[...] + matt(xr[...], wr)
        if with_relu:
            r = jnp.maximum(r, 0.0)
        o[...] = r

    row_spec = pl.BlockSpec((R, D), lambda i: (i, 0))
    p0_spec = pl.BlockSpec((1, R, D), lambda i: (0, i, 0))
    p1_spec = pl.BlockSpec((1, R, D), lambda i: (1, i, 0))
    col_spec = pl.BlockSpec((R, 1), lambda i: (i, 0))
    w_spec = pl.BlockSpec((D, D), lambda i: (0, 0))
    b_spec = pl.BlockSpec((1, D), lambda i: (0, 0))
    return pl.pallas_call(
        body,
        grid=(N_NODES // R,),
        in_specs=[p0_spec, p1_spec, col_spec, col_spec, row_spec,
                  w_spec, w_spec, b_spec],
        out_specs=row_spec,
        out_shape=jax.ShapeDtypeStruct((N_NODES, D), jnp.float32),
    )


_dense_relu = _make_dense(with_relu=True)
_dense = _make_dense(with_relu=False)


def kernel(x, edge_index, W1l, b1, W1r, W2l, b2, W2r):
    ei = edge_index.astype(jnp.int32)
    src4d = ei[0].reshape(NW, NB, BCH, CHUNK)
    dst4d = ei[1].reshape(NW, NB, BCH, CHUNK)

    part1, cnt0_f, cnt1_f = _sc_agg_cnt(x, src4d, dst4d)
    c0 = cnt0_f.reshape(NPAD, 1)
    c1 = cnt1_f.reshape(NPAD, 1)
    h = _dense_relu(part1, part1, c0, c1, x, W1l, W1r, b1.reshape(1, D))

    (part2,) = _sc_agg(h, src4d, dst4d)
    out = _dense(part2, part2, c0, c1, h, W2l, W2r, b2.reshape(1, D))
    return out


# speedup vs baseline: 1.1918x; 1.0634x over previous
import functools

import jax
import jax.numpy as jnp
from jax import lax
from jax.experimental import pallas as pl
from jax.experimental.pallas import tpu as pltpu
from jax.experimental.pallas import tpu_sc as plsc

N_NODES = 10000
N_EDGES = 320000
D = 128
NC = 2
NS = 16
NW = NC * NS
NPAD = 10240
RPT = NPAD // NS
CHUNK = 80
CPT = N_EDGES // CHUNK // NW
NB = 5
BCH = CPT // NB
LANES = 16


def _make_sc_agg(with_cnt: bool, nring: int = 2):
    out_type = [jax.ShapeDtypeStruct((NC, NPAD, D), jnp.float32)]
    if with_cnt:
        out_type.append(jax.ShapeDtypeStruct((NPAD,), jnp.float32))
        out_type.append(jax.ShapeDtypeStruct((NPAD,), jnp.float32))

    scratch = (
        [pltpu.VMEM((BCH, CHUNK), jnp.int32)] * 4 +
        [pltpu.VMEM((CHUNK, D), jnp.float32)] * nring +
        [pltpu.VMEM_SHARED((NPAD, D), jnp.float32)] +
        [pltpu.SemaphoreType.DMA] * (nring + 2)
    )
    if with_cnt:
        scratch += [
            pltpu.VMEM((RPT,), jnp.float32),
            pltpu.VMEM((CHUNK,), jnp.float32),
            pltpu.VMEM_SHARED((NPAD,), jnp.float32),
        ]

    def body(x_hbm, e5_hbm, part_out, *rest):
        if with_cnt:
            cnt_out0, cnt_out1 = rest[:2]
            rest = rest[2:]
            zc, ones, cnt_sh = rest[-3:]
        sidxA, didxA, sidxB, didxB = rest[:4]
        bufs = rest[4:4 + nring]
        acc = rest[4 + nring]
        sems = rest[5 + nring:5 + 2 * nring]
        semiA, semiB = rest[5 + 2 * nring:7 + 2 * nring]
        rows0 = bufs[0]
        cid = lax.axis_index("c")
        sid = lax.axis_index("s")
        w = sid * NC + cid
        zero16 = jnp.zeros((LANES,), jnp.float32)

        pltpu.async_copy(e5_hbm.at[0, w, 0], sidxA, semiA)
        pltpu.async_copy(e5_hbm.at[1, w, 0], didxA, semiA)

        def zrow(i, carry):
            for j in range(D // LANES):
                rows0[i, pl.ds(j * LANES, LANES)] = zero16
            return carry
        lax.fori_loop(0, CHUNK, zrow, 0)
        rb = sid * RPT
        for k in range(RPT // CHUNK):
            pltpu.sync_copy(rows0, acc.at[pl.ds(rb + k * CHUNK, CHUNK)])
        if with_cnt:
            def zrow2(i, carry):
                zc[pl.ds(i * LANES, LANES)] = zero16
                return carry
            lax.fori_loop(0, RPT // LANES, zrow2, 0)
            for j in range(CHUNK // LANES):
                ones[pl.ds(j * LANES, LANES)] = jnp.ones((LANES,), jnp.float32)
            pltpu.sync_copy(zc, cnt_sh.at[pl.ds(rb, RPT)])
        plsc.subcore_barrier()

        for b in range(NB):
            si, di, isem = ((sidxA, didxA, semiA) if b % 2 == 0
                            else (sidxB, didxB, semiB))
            pltpu.make_async_copy(e5_hbm.at[0, w, b], si, isem).wait()
            pltpu.make_async_copy(e5_hbm.at[1, w, b], di, isem).wait()
            if b + 1 < NB:
                nsi, ndi, nisem = ((sidxB, didxB, semiB) if b % 2 == 0
                                   else (sidxA, didxA, semiA))
                pltpu.async_copy(e5_hbm.at[0, w, b + 1], nsi, nisem)
                pltpu.async_copy(e5_hbm.at[1, w, b + 1], ndi, nisem)

            def scat(c, buf, di=di):
                pltpu.sync_copy(buf, acc.at[di.at[c]], add=True)
                if with_cnt:
                    pltpu.sync_copy(ones, cnt_sh.at[di.at[c]], add=True)

            for k in range(nring):
                pltpu.async_copy(x_hbm.at[si.at[k]], bufs[k], sems[k])

            def grp_body(t, carry, si=si, scat=scat):
                base = t * nring
                for k in range(nring):
                    c = base + k
                    pltpu.make_async_copy(x_hbm.at[si.at[c]], bufs[k],
                                          sems[k]).wait()
                    scat(c, bufs[k])
                    cn = jnp.minimum(c + nring, BCH - 1)
                    pltpu.async_copy(x_hbm.at[si.at[cn]], bufs[k], sems[k])
                return carry
            lax.fori_loop(0, (BCH - 1) // nring, grp_body, 0)
            pltpu.make_async_copy(x_hbm.at[si.at[BCH - 1]], bufs[0],
                                  sems[0]).wait()
            scat(BCH - 1, bufs[0])
            for k in range(1, nring):
                pltpu.make_async_copy(x_hbm.at[si.at[BCH - 1]], bufs[k],
                                      sems[k]).wait()

        plsc.subcore_barrier()

        pltpu.sync_copy(acc.at[pl.ds(rb, RPT)], part_out.at[cid, pl.ds(rb, RPT)])
        if with_cnt:
            @pl.when(cid == 0)
            def _():
                pltpu.sync_copy(cnt_sh.at[pl.ds(rb, RPT)], cnt_out0.at[pl.ds(rb, RPT)])

            @pl.when(cid == 1)
            def _():
                pltpu.sync_copy(cnt_sh.at[pl.ds(rb, RPT)], cnt_out1.at[pl.ds(rb, RPT)])

    mesh = plsc.VectorSubcoreMesh(core_axis_name="c", subcore_axis_name="s")
    return pl.kernel(body, out_type=tuple(out_type), mesh=mesh,
                     scratch_types=scratch)


_sc_agg_cnt = _make_sc_agg(with_cnt=True, nring=3)
_sc_agg = _make_sc_agg(with_cnt=False, nring=3)


def _make_dense(with_relu: bool):
    R = 2000

    def matt(a, w):
        return lax.dot_general(a, w[...], (((1,), (1,)), ((), ())),
                               preferred_element_type=jnp.float32)

    def body(p0, p1, c0, c1, xr, wl, wr, br, o):
        cnt = jnp.maximum(c0[...] + c1[...], 1.0)
        agg = (p0[0] + p1[0]) / cnt
        r = matt(agg, wl) + br[...] + matt(xr[...], wr)
        if with_relu:
            r = jnp.maximum(r, 0.0)
        o[...] = r

    row_spec = pl.BlockSpec((R, D), lambda i: (i, 0))
    p0_spec = pl.BlockSpec((1, R, D), lambda i: (0, i, 0))
    p1_spec = pl.BlockSpec((1, R, D), lambda i: (1, i, 0))
    col_spec = pl.BlockSpec((R, 1), lambda i: (i, 0))
    w_spec = pl.BlockSpec((D, D), lambda i: (0, 0))
    b_spec = pl.BlockSpec((1, D), lambda i: (0, 0))
    return pl.pallas_call(
        body,
        grid=(N_NODES // R,),
        in_specs=[p0_spec, p1_spec, col_spec, col_spec, row_spec,
                  w_spec, w_spec, b_spec],
        out_specs=row_spec,
        out_shape=jax.ShapeDtypeStruct((N_NODES, D), jnp.float32),
    )


_dense_relu = _make_dense(with_relu=True)
_dense = _make_dense(with_relu=False)


def kernel(x, edge_index, W1l, b1, W1r, W2l, b2, W2r):
    ei5 = edge_index.astype(jnp.int32).reshape(2, NW, NB, BCH, CHUNK)

    part1, cnt0_f, cnt1_f = _sc_agg_cnt(x, ei5)
    c0 = cnt0_f.reshape(NPAD, 1)
    c1 = cnt1_f.reshape(NPAD, 1)
    h = _dense_relu(part1, part1, c0, c1, x, W1l, W1r, b1.reshape(1, D))

    (part2,) = _sc_agg(h, ei5)
    out = _dense(part2, part2, c0, c1, h, W2l, W2r, b2.reshape(1, D))
    return out


# speedup vs baseline: 1.2029x; 1.0093x over previous
import functools

import jax
import jax.numpy as jnp
from jax import lax
from jax.experimental import pallas as pl
from jax.experimental.pallas import tpu as pltpu
from jax.experimental.pallas import tpu_sc as plsc

N_NODES = 10000
N_EDGES = 320000
D = 128
NC = 2
NS = 16
NW = NC * NS
NPAD = 10240
RPT = NPAD // NS
CHUNK = 80
CPT = N_EDGES // CHUNK // NW
NB = 5
BCH = CPT // NB
LANES = 16


def _make_sc_agg(with_cnt: bool, nring: int = 2):
    out_type = [jax.ShapeDtypeStruct((NC, NPAD, D), jnp.float32)]
    if with_cnt:
        out_type.append(jax.ShapeDtypeStruct((NPAD,), jnp.float32))
        out_type.append(jax.ShapeDtypeStruct((NPAD,), jnp.float32))

    scratch = (
        [pltpu.VMEM((BCH, CHUNK), jnp.int32)] * 4 +
        [pltpu.VMEM((CHUNK, D), jnp.float32)] * nring +
        [pltpu.VMEM_SHARED((NPAD, D), jnp.float32)] +
        [pltpu.SemaphoreType.DMA] * (nring + 2)
    )
    if with_cnt:
        scratch += [
            pltpu.VMEM((RPT,), jnp.float32),
            pltpu.VMEM((CHUNK,), jnp.float32),
            pltpu.VMEM_SHARED((NPAD,), jnp.float32),
        ]

    def body(x_hbm, e5_hbm, part_out, *rest):
        if with_cnt:
            cnt_out0, cnt_out1 = rest[:2]
            rest = rest[2:]
            zc, ones, cnt_sh = rest[-3:]
        sidxA, didxA, sidxB, didxB = rest[:4]
        bufs = rest[4:4 + nring]
        acc = rest[4 + nring]
        sems = rest[5 + nring:5 + 2 * nring]
        semiA, semiB = rest[5 + 2 * nring:7 + 2 * nring]
        rows0 = bufs[0]
        cid = lax.axis_index("c")
        sid = lax.axis_index("s")
        w = sid * NC + cid
        zero16 = jnp.zeros((LANES,), jnp.float32)

        pltpu.async_copy(e5_hbm.at[0, w, 0], sidxA, semiA)
        pltpu.async_copy(e5_hbm.at[1, w, 0], didxA, semiA)

        def zrow(i, carry):
            for j in range(D // LANES):
                rows0[i, pl.ds(j * LANES, LANES)] = zero16
            return carry
        lax.fori_loop(0, CHUNK, zrow, 0)
        rb = sid * RPT
        for k in range(RPT // CHUNK):
            pltpu.async_copy(rows0, acc.at[pl.ds(rb + k * CHUNK, CHUNK)],
                             sems[0])
        if with_cnt:
            def zrow2(i, carry):
                zc[pl.ds(i * LANES, LANES)] = zero16
                return carry
            lax.fori_loop(0, RPT // LANES, zrow2, 0)
            for j in range(CHUNK // LANES):
                ones[pl.ds(j * LANES, LANES)] = jnp.ones((LANES,), jnp.float32)
            pltpu.sync_copy(zc, cnt_sh.at[pl.ds(rb, RPT)])
        pltpu.make_async_copy(e5_hbm.at[0, w, 0], sidxA, semiA).wait()
        pltpu.make_async_copy(e5_hbm.at[1, w, 0], didxA, semiA).wait()
        for k in range(1, nring):
            pltpu.async_copy(x_hbm.at[sidxA.at[k]], bufs[k], sems[k])
        for k in range(RPT // CHUNK):
            pltpu.make_async_copy(rows0, acc.at[pl.ds(rb + k * CHUNK, CHUNK)],
                                  sems[0]).wait()
        pltpu.async_copy(x_hbm.at[sidxA.at[0]], bufs[0], sems[0])
        plsc.subcore_barrier()

        for b in range(NB):
            si, di, isem = ((sidxA, didxA, semiA) if b % 2 == 0
                            else (sidxB, didxB, semiB))
            if b > 0:
                pltpu.make_async_copy(e5_hbm.at[0, w, b], si, isem).wait()
                pltpu.make_async_copy(e5_hbm.at[1, w, b], di, isem).wait()
            if b + 1 < NB:
                nsi, ndi, nisem = ((sidxB, didxB, semiB) if b % 2 == 0
                                   else (sidxA, didxA, semiA))
                pltpu.async_copy(e5_hbm.at[0, w, b + 1], nsi, nisem)
                pltpu.async_copy(e5_hbm.at[1, w, b + 1], ndi, nisem)

            def scat(c, buf, di=di):
                pltpu.sync_copy(buf, acc.at[di.at[c]], add=True)
                if with_cnt:
                    pltpu.sync_copy(ones, cnt_sh.at[di.at[c]], add=True)

            if b > 0:
                for k in range(nring):
                    pltpu.async_copy(x_hbm.at[si.at[k]], bufs[k], sems[k])

            def grp_body(t, carry, si=si, scat=scat):
                base = t * nring
                for k in range(nring):
                    c = base + k
                    pltpu.make_async_copy(x_hbm.at[si.at[c]], bufs[k],
                                          sems[k]).wait()
                    scat(c, bufs[k])
                    cn = jnp.minimum(c + nring, BCH - 1)
                    pltpu.async_copy(x_hbm.at[si.at[cn]], bufs[k], sems[k])
                return carry
            lax.fori_loop(0, (BCH - 1) // nring, grp_body, 0)
            pltpu.make_async_copy(x_hbm.at[si.at[BCH - 1]], bufs[0],
                                  sems[0]).wait()
            scat(BCH - 1, bufs[0])
            for k in range(1, nring):
                pltpu.make_async_copy(x_hbm.at[si.at[BCH - 1]], bufs[k],
                                      sems[k]).wait()

        plsc.subcore_barrier()

        pltpu.sync_copy(acc.at[pl.ds(rb, RPT)], part_out.at[cid, pl.ds(rb, RPT)])
        if with_cnt:
            @pl.when(cid == 0)
            def _():
                pltpu.sync_copy(cnt_sh.at[pl.ds(rb, RPT)], cnt_out0.at[pl.ds(rb, RPT)])

            @pl.when(cid == 1)
            def _():
                pltpu.sync_copy(cnt_sh.at[pl.ds(rb, RPT)], cnt_out1.at[pl.ds(rb, RPT)])

    mesh = plsc.VectorSubcoreMesh(core_axis_name="c", subcore_axis_name="s")
    return pl.kernel(body, out_type=tuple(out_type), mesh=mesh,
                     scratch_types=scratch)


_sc_agg_cnt = _make_sc_agg(with_cnt=True, nring=3)
_sc_agg = _make_sc_agg(with_cnt=False, nring=3)


def _make_dense(with_relu: bool):
    R = 2000

    def matt(a, w):
        return lax.dot_general(a, w[...], (((1,), (1,)), ((), ())),
                               preferred_element_type=jnp.float32)

    def body(p0, p1, c0, c1, xr, wl, wr, br, o):
        cnt = jnp.maximum(c0[...] + c1[...], 1.0)
        agg = (p0[0] + p1[0]) / cnt
        r = matt(agg, wl) + br[...] + matt(xr[...], wr)
        if with_relu:
            r = jnp.maximum(r, 0.0)
        o[...] = r

    row_spec = pl.BlockSpec((R, D), lambda i: (i, 0))
    p0_spec = pl.BlockSpec((1, R, D), lambda i: (0, i, 0))
    p1_spec = pl.BlockSpec((1, R, D), lambda i: (1, i, 0))
    col_spec = pl.BlockSpec((R, 1), lambda i: (i, 0))
    w_spec = pl.BlockSpec((D, D), lambda i: (0, 0))
    b_spec = pl.BlockSpec((1, D), lambda i: (0, 0))
    return pl.pallas_call(
        body,
        grid=(N_NODES // R,),
        in_specs=[p0_spec, p1_spec, col_spec, col_spec, row_spec,
                  w_spec, w_spec, b_spec],
        out_specs=row_spec,
        out_shape=jax.ShapeDtypeStruct((N_NODES, D), jnp.float32),
    )


_dense_relu = _make_dense(with_relu=True)
_dense = _make_dense(with_relu=False)


def kernel(x, edge_index, W1l, b1, W1r, W2l, b2, W2r):
    ei5 = edge_index.astype(jnp.int32).reshape(2, NW, NB, BCH, CHUNK)

    part1, cnt0_f, cnt1_f = _sc_agg_cnt(x, ei5)
    c0 = cnt0_f.reshape(NPAD, 1)
    c1 = cnt1_f.reshape(NPAD, 1)
    h = _dense_relu(part1, part1, c0, c1, x, W1l, W1r, b1.reshape(1, D))

    (part2,) = _sc_agg(h, ei5)
    out = _dense(part2, part2, c0, c1, h, W2l, W2r, b2.reshape(1, D))
    return out


# speedup vs baseline: 1.2955x; 1.0770x over previous
import functools

import jax
import jax.numpy as jnp
from jax import lax
from jax.experimental import pallas as pl
from jax.experimental.pallas import tpu as pltpu
from jax.experimental.pallas import tpu_sc as plsc

N_NODES = 10000
N_EDGES = 320000
D = 128
NC = 2
NS = 16
NW = NC * NS
NPAD = 10240
RPT = NPAD // NS
CHUNK = 80
CPT = N_EDGES // CHUNK // NW
NB = 5
BCH = CPT // NB
LANES = 16


def _make_sc_agg(with_cnt: bool, nring: int = 2):
    out_type = [jax.ShapeDtypeStruct((NC, NPAD, D), jnp.float32)]
    if with_cnt:
        out_type.append(jax.ShapeDtypeStruct((NPAD,), jnp.float32))
        out_type.append(jax.ShapeDtypeStruct((NPAD,), jnp.float32))

    scratch = (
        [pltpu.VMEM((BCH, CHUNK), jnp.int32)] * 4 +
        [pltpu.VMEM((CHUNK, D), jnp.float32)] * nring +
        [pltpu.VMEM_SHARED((NPAD, D), jnp.float32)] +
        [pltpu.SemaphoreType.DMA] * (nring + 2)
    )
    if with_cnt:
        scratch += [
            pltpu.VMEM((RPT,), jnp.float32),
            pltpu.VMEM((CHUNK,), jnp.float32),
            pltpu.VMEM_SHARED((NPAD,), jnp.float32),
        ]

    def body(x_hbm, e5_hbm, part_out, *rest):
        if with_cnt:
            cnt_out0, cnt_out1 = rest[:2]
            rest = rest[2:]
            zc, ones, cnt_sh = rest[-3:]
        sidxA, didxA, sidxB, didxB = rest[:4]
        bufs = rest[4:4 + nring]
        acc = rest[4 + nring]
        sems = rest[5 + nring:5 + 2 * nring]
        semiA, semiB = rest[5 + 2 * nring:7 + 2 * nring]
        rows0 = bufs[0]
        cid = lax.axis_index("c")
        sid = lax.axis_index("s")
        w = sid * NC + cid
        zero16 = jnp.zeros((LANES,), jnp.float32)

        pltpu.async_copy(e5_hbm.at[0, w, 0], sidxA, semiA)
        pltpu.async_copy(e5_hbm.at[1, w, 0], didxA, semiA)

        def zrow(i, carry):
            for j in range(D // LANES):
                rows0[i, pl.ds(j * LANES, LANES)] = zero16
            return carry
        lax.fori_loop(0, CHUNK, zrow, 0)
        rb = sid * RPT
        for k in range(RPT // CHUNK):
            pltpu.async_copy(rows0, acc.at[pl.ds(rb + k * CHUNK, CHUNK)],
                             sems[0])
        if with_cnt:
            def zrow2(i, carry):
                zc[pl.ds(i * LANES, LANES)] = zero16
                return carry
            lax.fori_loop(0, RPT // LANES, zrow2, 0)
            for j in range(CHUNK // LANES):
                ones[pl.ds(j * LANES, LANES)] = jnp.ones((LANES,), jnp.float32)
            pltpu.sync_copy(zc, cnt_sh.at[pl.ds(rb, RPT)])
        pltpu.make_async_copy(e5_hbm.at[0, w, 0], sidxA, semiA).wait()
        pltpu.make_async_copy(e5_hbm.at[1, w, 0], didxA, semiA).wait()
        for k in range(1, nring):
            pltpu.async_copy(x_hbm.at[sidxA.at[k]], bufs[k], sems[k])
        for k in range(RPT // CHUNK):
            pltpu.make_async_copy(rows0, acc.at[pl.ds(rb + k * CHUNK, CHUNK)],
                                  sems[0]).wait()
        pltpu.async_copy(x_hbm.at[sidxA.at[0]], bufs[0], sems[0])
        plsc.subcore_barrier()

        assert BCH % nring == 1
        for b in range(NB):
            si, di, isem = ((sidxA, didxA, semiA) if b % 2 == 0
                            else (sidxB, didxB, semiB))
            nsi, ndi, nisem = ((sidxB, didxB, semiB) if b % 2 == 0
                               else (sidxA, didxA, semiA))
            if b + 1 < NB:
                pltpu.async_copy(e5_hbm.at[0, w, b + 1], nsi, nisem)
                pltpu.async_copy(e5_hbm.at[1, w, b + 1], ndi, nisem)

            def scat(c, buf, di=di):
                pltpu.sync_copy(buf, acc.at[di.at[c]], add=True)
                if with_cnt:
                    pltpu.sync_copy(ones, cnt_sh.at[di.at[c]], add=True)

            def grp_body(t, carry, si=si, scat=scat, b=b):
                for k in range(nring):
                    c = t * nring + k
                    buf, sem = bufs[(b + k) % nring], sems[(b + k) % nring]
                    pltpu.make_async_copy(x_hbm.at[si.at[c]], buf, sem).wait()
                    scat(c, buf)
                    pltpu.async_copy(x_hbm.at[si.at[c + nring]], buf, sem)
                return carry
            lax.fori_loop(0, (BCH - 1) // nring - 1, grp_body, 0)
            if b + 1 < NB:
                pltpu.make_async_copy(e5_hbm.at[0, w, b + 1], nsi, nisem).wait()
                pltpu.make_async_copy(e5_hbm.at[1, w, b + 1], ndi, nisem).wait()
            for j in range(BCH - 1 - nring, BCH):
                buf, sem = bufs[(b + j) % nring], sems[(b + j) % nring]
                pltpu.make_async_copy(x_hbm.at[si.at[j]], buf, sem).wait()
                scat(j, buf)
                if j + nring < BCH:
                    pltpu.async_copy(x_hbm.at[si.at[j + nring]], buf, sem)
                elif b + 1 < NB:
                    pltpu.async_copy(x_hbm.at[nsi.at[j + nring - BCH]], buf, sem)
                else:
                    pltpu.async_copy(x_hbm.at[si.at[BCH - 1]], buf, sem)

        si_last = sidxA if (NB - 1) % 2 == 0 else sidxB
        for k in range(nring):
            pltpu.make_async_copy(x_hbm.at[si_last.at[BCH - 1]], bufs[k],
                                  sems[k]).wait()

        plsc.subcore_barrier()

        pltpu.sync_copy(acc.at[pl.ds(rb, RPT)], part_out.at[cid, pl.ds(rb, RPT)])
        if with_cnt:
            @pl.when(cid == 0)
            def _():
                pltpu.sync_copy(cnt_sh.at[pl.ds(rb, RPT)], cnt_out0.at[pl.ds(rb, RPT)])

            @pl.when(cid == 1)
            def _():
                pltpu.sync_copy(cnt_sh.at[pl.ds(rb, RPT)], cnt_out1.at[pl.ds(rb, RPT)])

    mesh = plsc.VectorSubcoreMesh(core_axis_name="c", subcore_axis_name="s")
    return pl.kernel(body, out_type=tuple(out_type), mesh=mesh,
                     scratch_types=scratch)


_sc_agg_cnt = _make_sc_agg(with_cnt=True, nring=3)
_sc_agg = _make_sc_agg(with_cnt=False, nring=3)


def _make_dense(with_relu: bool):
    R = 2000

    def matt(a, w):
        return lax.dot_general(a, w[...], (((1,), (1,)), ((), ())),
                               preferred_element_type=jnp.float32)

    def body(p0, p1, c0, c1, xr, wl, wr, br, o):
        cnt = jnp.maximum(c0[...] + c1[...], 1.0)
        agg = (p0[0] + p1[0]) / cnt
        r = matt(agg, wl) + br[...] + matt(xr[...], wr)
        if with_relu:
            r = jnp.maximum(r, 0.0)
        o[...] = r

    row_spec = pl.BlockSpec((R, D), lambda i: (i, 0))
    p0_spec = pl.BlockSpec((1, R, D), lambda i: (0, i, 0))
    p1_spec = pl.BlockSpec((1, R, D), lambda i: (1, i, 0))
    col_spec = pl.BlockSpec((R, 1), lambda i: (i, 0))
    w_spec = pl.BlockSpec((D, D), lambda i: (0, 0))
    b_spec = pl.BlockSpec((1, D), lambda i: (0, 0))
    return pl.pallas_call(
        body,
        grid=(N_NODES // R,),
        in_specs=[p0_spec, p1_spec, col_spec, col_spec, row_spec,
                  w_spec, w_spec, b_spec],
        out_specs=row_spec,
        out_shape=jax.ShapeDtypeStruct((N_NODES, D), jnp.float32),
    )


_dense_relu = _make_dense(with_relu=True)
_dense = _make_dense(with_relu=False)


def kernel(x, edge_index, W1l, b1, W1r, W2l, b2, W2r):
    ei5 = edge_index.astype(jnp.int32).reshape(2, NW, NB, BCH, CHUNK)

    part1, cnt0_f, cnt1_f = _sc_agg_cnt(x, ei5)
    c0 = cnt0_f.reshape(NPAD, 1)
    c1 = cnt1_f.reshape(NPAD, 1)
    h = _dense_relu(part1, part1, c0, c1, x, W1l, W1r, b1.reshape(1, D))

    (part2,) = _sc_agg(h, ei5)
    out = _dense(part2, part2, c0, c1, h, W2l, W2r, b2.reshape(1, D))
    return out
